# Initial kernel scaffold; baseline (speedup 1.0000x reference)
import jax
import jax.numpy as jnp
from jax.experimental import pallas as pl


def kernel(src_x, src_xyz, xyz, W0, W1, W2, g0, b0, g1, b1, gN, bN):
    raise NotImplementedError("write your pallas kernel here")


# speedup vs baseline: 108.6673x; 108.6673x over previous
import functools
import math

import jax
import jax.numpy as jnp
from jax.experimental import pallas as pl
from jax.experimental.pallas import tpu as pltpu
from jax.experimental.pallas import tpu_sc as plsc

_B, _N, _M = 2, 8192, 2048
_K = 32
_RADIUS2 = 0.2 * 0.2
_EPS = 1e-5

_TM = 256
_C = 1024
_TRK = 256
_GW = 128


def _gelu(x):
    return 0.5 * x * (1.0 + jax.lax.erf(x * (1.0 / math.sqrt(2.0))))


def _cumsum_lanes(x):
    c = x.shape[-1]
    ii = jax.lax.broadcasted_iota(jnp.int32, x.shape, len(x.shape) - 1)
    s = 1
    while s < c:
        sh = pltpu.roll(x, shift=s, axis=len(x.shape) - 1)
        x = x + jnp.where(ii >= s, sh, 0)
        s *= 2
    return x


def _k0_body(sx_ref, sxyz_ref, q_ref, w_ref, tt_ref, qt_ref):
    w = w_ref[...]
    wx = w[:, :64]
    wz = w[:, 64:67]
    sx = sx_ref[0]
    sxyz = sxyz_ref[0]
    q = q_ref[0]
    dn = (((0,), (1,)), ((), ()))
    hi = jax.lax.Precision.HIGHEST
    t = jax.lax.dot_general(sx, wx, dn, precision=hi,
                            preferred_element_type=jnp.float32)
    t = t + jax.lax.dot_general(sxyz, wz, dn, precision=hi,
                                preferred_element_type=jnp.float32)
    tt_ref[0] = jnp.concatenate([t, jnp.zeros((_N, 64), jnp.float32)], axis=1)
    qt_ref[0] = jax.lax.dot_general(q, wz, dn, precision=hi,
                                    preferred_element_type=jnp.float32)


def _k0(src_x, src_xyz, xyz, w0):
    return pl.pallas_call(
        _k0_body,
        grid=(_B,),
        in_specs=[
            pl.BlockSpec((1, 64, _N), lambda b: (b, 0, 0)),
            pl.BlockSpec((1, 3, _N), lambda b: (b, 0, 0)),
            pl.BlockSpec((1, 3, _M), lambda b: (b, 0, 0)),
            pl.BlockSpec((64, 67), lambda b: (0, 0)),
        ],
        out_specs=[
            pl.BlockSpec((1, _N, 128), lambda b: (b, 0, 0)),
            pl.BlockSpec((1, _M, 64), lambda b: (b, 0, 0)),
        ],
        out_shape=[
            jax.ShapeDtypeStruct((_B, _N, 128), jnp.float32),
            jax.ShapeDtypeStruct((_B, _M, 64), jnp.float32),
        ],
    )(src_x, src_xyz, xyz, w0)


def _k1_body(q_ref, s_ref, idx_ref):
    b = pl.program_id(0)
    q = q_ref[0]
    qx = q[:, 0:1]
    qy = q[:, 1:2]
    qz = q[:, 2:3]
    q2 = qx * qx + qy * qy + qz * qz
    qxb = qx.astype(jnp.bfloat16).astype(jnp.float32)
    qyb = qy.astype(jnp.bfloat16).astype(jnp.float32)
    qzb = qz.astype(jnp.bfloat16).astype(jnp.float32)

    def chunk(ci, carry):
        cnt, acc = carry
        s = s_ref[0, :, pl.ds(ci * _C, _C)]
        sx = s[0:1, :]
        sy = s[1:2, :]
        sz = s[2:3, :]
        s2 = sx * sx + sy * sy + sz * sz
        sxb = sx.astype(jnp.bfloat16).astype(jnp.float32)
        syb = sy.astype(jnp.bfloat16).astype(jnp.float32)
        szb = sz.astype(jnp.bfloat16).astype(jnp.float32)
        dot = qxb * sxb + qyb * syb + qzb * szb
        d2 = (q2 + s2) - 2.0 * dot
        mask = d2 <= _RADIUS2
        mi = mask.astype(jnp.int32)
        cum = _cumsum_lanes(mi)
        p = jnp.where(mask, cnt + cum, 0)
        ng = jax.lax.broadcasted_iota(jnp.int32, (_TM, _C), 1) + ci * _C
        cols = []
        for j in range(_K):
            cols.append(jnp.sum(jnp.where(p == (j + 1), ng, 0),
                                axis=1, keepdims=True))
        acc = acc + jnp.concatenate(cols, axis=1)
        cnt = cnt + cum[:, _C - 1:_C]
        return cnt, acc

    cnt0 = jnp.zeros((_TM, 1), jnp.int32)
    acc0 = jnp.zeros((_TM, _K), jnp.int32)
    cnt, acc = jax.lax.fori_loop(0, _N // _C, chunk, (cnt0, acc0))

    jvec = jax.lax.broadcasted_iota(jnp.int32, (_TM, _K), 1)
    filled = jnp.minimum(cnt, _K)
    idx = jnp.where(jvec < filled, acc, acc[:, 0:1])
    idx_ref[0] = idx + b * _N


def _k1(xyzt, src_xyz):
    return pl.pallas_call(
        _k1_body,
        grid=(_B, _M // _TM),
        in_specs=[
            pl.BlockSpec((1, _TM, 3), lambda b, t: (b, t, 0)),
            pl.BlockSpec((1, 3, _N), lambda b, t: (b, 0, 0)),
        ],
        out_specs=pl.BlockSpec((1, _TM, _K), lambda b, t: (b, t, 0)),
        out_shape=jax.ShapeDtypeStruct((_B, _M, _K), jnp.int32),
    )(xyzt, src_xyz)


def _gather_rows(table, indices):
    num = indices.shape[1]
    mesh = plsc.VectorSubcoreMesh(core_axis_name="c", subcore_axis_name="s")

    @pl.kernel(out_type=jax.ShapeDtypeStruct((num, table.shape[1]),
                                             table.dtype),
               mesh=mesh)
    def k(tab_hbm, i_hbm, o_hbm):
        def body(i_vmem, o_vmem):
            pltpu.sync_copy(tab_hbm.at[i_vmem.at[0]], o_vmem)

        pltpu.emit_pipeline(
            body,
            grid=(num // _GW,),
            in_specs=[pl.BlockSpec((1, _GW), index_map=lambda i: (0, i))],
            out_specs=[pl.BlockSpec((_GW, table.shape[1]),
                                    index_map=lambda i: (i, 0))],
            core_axis_name=("c", "s"),
            dimension_semantics=(pltpu.PARALLEL,),
        )(i_hbm, o_hbm)

    return k(table, indices)


def _k3_body(g_ref, q_ref, st_ref):
    i = pl.program_id(0)
    g = g_ref[...][:, :, :64]
    q = q_ref[...]
    x = g - q[:, None, :]
    s = jnp.sum(x, axis=(0, 1))
    s2 = jnp.sum(x * x, axis=(0, 1))
    blk = jnp.concatenate(
        [s[None, :], s2[None, :], jnp.zeros((6, s.shape[0]), jnp.float32)],
        axis=0)

    @pl.when(i == 0)
    def _():
        st_ref[...] = blk

    @pl.when(i > 0)
    def _():
        st_ref[...] += blk


def _k3(g3, qf):
    nq = g3.shape[0]
    return pl.pallas_call(
        _k3_body,
        grid=(nq // _TRK,),
        in_specs=[
            pl.BlockSpec((_TRK, _K, 128), lambda i: (i, 0, 0)),
            pl.BlockSpec((_TRK, 64), lambda i: (i, 0)),
        ],
        out_specs=pl.BlockSpec((8, 64), lambda i: (0, 0)),
        out_shape=jax.ShapeDtypeStruct((8, 64), jnp.float32),
    )(g3, qf)


def _k4_body(g_ref, q_ref, ac_ref, w_ref, x2_ref, st_ref):
    i = pl.program_id(0)
    g = g_ref[...][:, :, :64]
    q = q_ref[...]
    a = ac_ref[0:1, :][None]
    c = ac_ref[1:2, :][None]
    x1 = g - q[:, None, :]
    y1 = _gelu(x1 * a + c)
    dn = (((2,), (1,)), ((), ()))
    x2 = jax.lax.dot_general(y1, w_ref[...], dn,
                             precision=jax.lax.Precision.HIGHEST,
                             preferred_element_type=jnp.float32)
    x2_ref[...] = x2
    s = jnp.sum(x2, axis=(0, 1))
    s2 = jnp.sum(x2 * x2, axis=(0, 1))
    blk = jnp.concatenate(
        [s[None, :], s2[None, :], jnp.zeros((6, s.shape[0]), jnp.float32)],
        axis=0)

    @pl.when(i == 0)
    def _():
        st_ref[...] = blk

    @pl.when(i > 0)
    def _():
        st_ref[...] += blk


def _k4(g3, qf, ac0, w1):
    nq = g3.shape[0]
    return pl.pallas_call(
        _k4_body,
        grid=(nq // _TRK,),
        in_specs=[
            pl.BlockSpec((_TRK, _K, 128), lambda i: (i, 0, 0)),
            pl.BlockSpec((_TRK, 64), lambda i: (i, 0)),
            pl.BlockSpec((8, 64), lambda i: (0, 0)),
            pl.BlockSpec((64, 64), lambda i: (0, 0)),
        ],
        out_specs=[
            pl.BlockSpec((_TRK, _K, 64), lambda i: (i, 0, 0)),
            pl.BlockSpec((8, 64), lambda i: (0, 0)),
        ],
        out_shape=[
            jax.ShapeDtypeStruct((nq, _K, 64), jnp.float32),
            jax.ShapeDtypeStruct((8, 64), jnp.float32),
        ],
    )(g3, qf, ac0, w1)


def _k5_body(x2_ref, ac_ref, w_ref, xm_ref, st_ref):
    i = pl.program_id(0)
    x2 = x2_ref[...]
    a = ac_ref[0:1, :][None]
    c = ac_ref[1:2, :][None]
    y2 = _gelu(x2 * a + c)
    dn = (((2,), (1,)), ((), ()))
    x3 = jax.lax.dot_general(y2, w_ref[...], dn,
                             precision=jax.lax.Precision.HIGHEST,
                             preferred_element_type=jnp.float32)
    xm = jnp.max(x3, axis=1)
    xm_ref[...] = xm
    s = jnp.sum(xm, axis=0)
    s2 = jnp.sum(xm * xm, axis=0)
    blk = jnp.concatenate(
        [s[None, :], s2[None, :], jnp.zeros((6, s.shape[0]), jnp.float32)],
        axis=0)

    @pl.when(i == 0)
    def _():
        st_ref[...] = blk

    @pl.when(i > 0)
    def _():
        st_ref[...] += blk


def _k5(x2, ac1, w2):
    nq = x2.shape[0]
    return pl.pallas_call(
        _k5_body,
        grid=(nq // _TRK,),
        in_specs=[
            pl.BlockSpec((_TRK, _K, 64), lambda i: (i, 0, 0)),
            pl.BlockSpec((8, 64), lambda i: (0, 0)),
            pl.BlockSpec((128, 64), lambda i: (0, 0)),
        ],
        out_specs=[
            pl.BlockSpec((_TRK, 128), lambda i: (i, 0)),
            pl.BlockSpec((8, 128), lambda i: (0, 0)),
        ],
        out_shape=[
            jax.ShapeDtypeStruct((nq, 128), jnp.float32),
            jax.ShapeDtypeStruct((8, 128), jnp.float32),
        ],
    )(x2, ac1, w2)


def _k6_body(xm_ref, ac_ref, o_ref):
    x = xm_ref[0]
    a = ac_ref[0:1, :]
    c = ac_ref[1:2, :]
    y = _gelu(x * a + c)
    o_ref[0] = jnp.transpose(y)


def _k6(xm3, acn):
    tmo = 512
    return pl.pallas_call(
        _k6_body,
        grid=(_B, _M // tmo),
        in_specs=[
            pl.BlockSpec((1, tmo, 128), lambda b, t: (b, t, 0)),
            pl.BlockSpec((8, 128), lambda b, t: (0, 0)),
        ],
        out_specs=pl.BlockSpec((1, 128, tmo), lambda b, t: (b, 0, t)),
        out_shape=jax.ShapeDtypeStruct((_B, 128, _M), jnp.float32),
    )(xm3, acn)


def _affine(stats, gamma, beta, n):
    s = stats[0]
    s2 = stats[1]
    mean = s / n
    var = s2 / n - mean * mean
    a = gamma * jax.lax.rsqrt(var + _EPS)
    c = beta - mean * a
    pad = jnp.zeros((6, gamma.shape[0]), jnp.float32)
    return jnp.concatenate([a[None, :], c[None, :], pad], axis=0)


def kernel(src_x, src_xyz, xyz, W0, W1, W2, g0, b0, g1, b1, gN, bN):
    tt, qt = _k0(src_x, src_xyz, xyz, W0)
    xyzt = jnp.transpose(xyz, (0, 2, 1))
    idx = _k1(xyzt, src_xyz)
    table = tt.reshape(_B * _N, 128)
    flat_idx = idx.reshape(1, _B * _M * _K)
    g = _gather_rows(table, flat_idx)
    g3 = g.reshape(_B * _M, _K, 128)
    qf = qt.reshape(_B * _M, 64)

    n0 = float(_B * _M * _K)
    st0 = _k3(g3, qf)
    ac0 = _affine(st0, g0, b0, n0)
    x2, st1 = _k4(g3, qf, ac0, W1)
    ac1 = _affine(st1, g1, b1, n0)
    xm, stn = _k5(x2, ac1, W2)
    acn = _affine(stn, gN, bN, float(_B * _M))
    xm3 = xm.reshape(_B, _M, 128)
    return _k6(xm3, acn)


# speedup vs baseline: 115.3605x; 1.0616x over previous
import math

import jax
import jax.numpy as jnp
from jax.experimental import pallas as pl
from jax.experimental.pallas import tpu as pltpu
from jax.experimental.pallas import tpu_sc as plsc

_B, _N, _M = 2, 8192, 2048
_K = 32
_RADIUS2 = 0.2 * 0.2
_EPS = 1e-5

_TM = 256
_C = 1024
_TRK = 256
_GW = 128


def _gelu(x):
    return 0.5 * x * (1.0 + jax.lax.erf(x * (1.0 / math.sqrt(2.0))))


def _cumsum_lanes(x):
    c = x.shape[-1]
    ii = jax.lax.broadcasted_iota(jnp.int32, x.shape, len(x.shape) - 1)
    s = 1
    while s < c:
        sh = pltpu.roll(x, shift=s, axis=len(x.shape) - 1)
        x = x + jnp.where(ii >= s, sh, 0)
        s *= 2
    return x


def _bf16_dot(lhs, w):
    dn = (((lhs.ndim - 1,), (1,)), ((), ()))
    return jax.lax.dot_general(lhs.astype(jnp.bfloat16),
                               w.astype(jnp.bfloat16), dn,
                               preferred_element_type=jnp.float32)


_NT = 2048


def _k0_body(sx_ref, sxyz_ref, tab_ref):
    xt = jnp.transpose(sx_ref[0])
    zt = jnp.transpose(sxyz_ref[0])
    tab_ref[0] = jnp.concatenate(
        [xt, zt, jnp.zeros((_NT, 61), jnp.float32)], axis=1)


def _k0(src_x, src_xyz):
    return pl.pallas_call(
        _k0_body,
        grid=(_B, _N // _NT),
        in_specs=[
            pl.BlockSpec((1, 64, _NT), lambda b, t: (b, 0, t)),
            pl.BlockSpec((1, 3, _NT), lambda b, t: (b, 0, t)),
        ],
        out_specs=pl.BlockSpec((1, _NT, 128), lambda b, t: (b, t, 0)),
        out_shape=jax.ShapeDtypeStruct((_B, _N, 128), jnp.float32),
    )(src_x, src_xyz)


def _k1_body(q_ref, s_ref, idx_ref):
    b = pl.program_id(0)
    q = q_ref[0]
    qx = q[:, 0:1]
    qy = q[:, 1:2]
    qz = q[:, 2:3]
    q2 = qx * qx + qy * qy + qz * qz
    qxb = qx.astype(jnp.bfloat16).astype(jnp.float32)
    qyb = qy.astype(jnp.bfloat16).astype(jnp.float32)
    qzb = qz.astype(jnp.bfloat16).astype(jnp.float32)

    def chunk(ci, carry):
        cnt, acc = carry
        s = s_ref[0, :, pl.ds(ci * _C, _C)]
        sx = s[0:1, :]
        sy = s[1:2, :]
        sz = s[2:3, :]
        s2 = sx * sx + sy * sy + sz * sz
        sxb = sx.astype(jnp.bfloat16).astype(jnp.float32)
        syb = sy.astype(jnp.bfloat16).astype(jnp.float32)
        szb = sz.astype(jnp.bfloat16).astype(jnp.float32)
        dot = qxb * sxb + qyb * syb + qzb * szb
        d2 = (q2 + s2) - 2.0 * dot
        mask = d2 <= _RADIUS2
        mi = mask.astype(jnp.int32)
        cum = _cumsum_lanes(mi)
        p = jnp.where(mask, cnt + cum, 0)
        ng = jax.lax.broadcasted_iota(jnp.int32, (_TM, _C), 1) + ci * _C
        cols = []
        for j in range(_K):
            cols.append(jnp.sum(jnp.where(p == (j + 1), ng, 0),
                                axis=1, keepdims=True))
        acc = acc + jnp.concatenate(cols, axis=1)
        cnt = cnt + cum[:, _C - 1:_C]
        return cnt, acc

    cnt0 = jnp.zeros((_TM, 1), jnp.int32)
    acc0 = jnp.zeros((_TM, _K), jnp.int32)
    cnt, acc = jax.lax.fori_loop(0, _N // _C, chunk, (cnt0, acc0))

    jvec = jax.lax.broadcasted_iota(jnp.int32, (_TM, _K), 1)
    filled = jnp.minimum(cnt, _K)
    idx = jnp.where(jvec < filled, acc, acc[:, 0:1])
    idx_ref[0] = idx + b * _N


def _k1(xyzt, src_xyz):
    return pl.pallas_call(
        _k1_body,
        grid=(_B, _M // _TM),
        in_specs=[
            pl.BlockSpec((1, _TM, 3), lambda b, t: (b, t, 0)),
            pl.BlockSpec((1, 3, _N), lambda b, t: (b, 0, 0)),
        ],
        out_specs=pl.BlockSpec((1, _TM, _K), lambda b, t: (b, t, 0)),
        out_shape=jax.ShapeDtypeStruct((_B, _M, _K), jnp.int32),
    )(xyzt, src_xyz)


def _gather_rows(table, indices):
    num = indices.shape[1]
    mesh = plsc.VectorSubcoreMesh(core_axis_name="c", subcore_axis_name="s")

    @pl.kernel(out_type=jax.ShapeDtypeStruct((num, table.shape[1]),
                                             table.dtype),
               mesh=mesh)
    def k(tab_hbm, i_hbm, o_hbm):
        def body(i_vmem, o_vmem):
            pltpu.sync_copy(tab_hbm.at[i_vmem.at[0]], o_vmem)

        pltpu.emit_pipeline(
            body,
            grid=(num // _GW,),
            in_specs=[pl.BlockSpec((1, _GW), index_map=lambda i: (0, i))],
            out_specs=[pl.BlockSpec((_GW, table.shape[1]),
                                    index_map=lambda i: (i, 0))],
            core_axis_name=("c", "s"),
            dimension_semantics=(pltpu.PARALLEL,),
        )(i_hbm, o_hbm)

    return k(table, indices)


def _x1_from_gathered(g_all, q, w0):
    xg = g_all[:, :, :64]
    sg = g_all[:, :, 64:67]
    nx = sg - q[:, None, :]
    feat = jnp.concatenate([xg, nx], axis=2)
    return _bf16_dot(feat, w0)


def _stats_rows(x):
    axes = tuple(range(x.ndim - 1))
    s = jnp.sum(x, axis=axes)
    s2 = jnp.sum(x * x, axis=axes)
    return jnp.concatenate(
        [s[None, :], s2[None, :], jnp.zeros((6, s.shape[0]), jnp.float32)],
        axis=0)


def _k3_body(g_ref, q_ref, w0_ref, st_ref):
    i = pl.program_id(0)
    x1 = _x1_from_gathered(g_ref[...], q_ref[...], w0_ref[...])
    blk = _stats_rows(x1)

    @pl.when(i == 0)
    def _():
        st_ref[...] = blk

    @pl.when(i > 0)
    def _():
        st_ref[...] += blk


def _k3(g3, qf, w0):
    nq = g3.shape[0]
    return pl.pallas_call(
        _k3_body,
        grid=(nq // _TRK,),
        in_specs=[
            pl.BlockSpec((_TRK, _K, 128), lambda i: (i, 0, 0)),
            pl.BlockSpec((_TRK, 3), lambda i: (i, 0)),
            pl.BlockSpec((64, 67), lambda i: (0, 0)),
        ],
        out_specs=pl.BlockSpec((8, 64), lambda i: (0, 0)),
        out_shape=jax.ShapeDtypeStruct((8, 64), jnp.float32),
    )(g3, qf, w0)


def _k4_body(g_ref, q_ref, w0_ref, ac_ref, w1_ref, x2_ref, st_ref):
    i = pl.program_id(0)
    x1 = _x1_from_gathered(g_ref[...], q_ref[...], w0_ref[...])
    a = ac_ref[0:1, :][None]
    c = ac_ref[1:2, :][None]
    y1 = _gelu(x1 * a + c)
    x2 = _bf16_dot(y1, w1_ref[...])
    x2_ref[...] = x2
    blk = _stats_rows(x2)

    @pl.when(i == 0)
    def _():
        st_ref[...] = blk

    @pl.when(i > 0)
    def _():
        st_ref[...] += blk


def _k4(g3, qf, w0, ac0, w1):
    nq = g3.shape[0]
    return pl.pallas_call(
        _k4_body,
        grid=(nq // _TRK,),
        in_specs=[
            pl.BlockSpec((_TRK, _K, 128), lambda i: (i, 0, 0)),
            pl.BlockSpec((_TRK, 3), lambda i: (i, 0)),
            pl.BlockSpec((64, 67), lambda i: (0, 0)),
            pl.BlockSpec((8, 64), lambda i: (0, 0)),
            pl.BlockSpec((64, 64), lambda i: (0, 0)),
        ],
        out_specs=[
            pl.BlockSpec((_TRK, _K, 64), lambda i: (i, 0, 0)),
            pl.BlockSpec((8, 64), lambda i: (0, 0)),
        ],
        out_shape=[
            jax.ShapeDtypeStruct((nq, _K, 64), jnp.float32),
            jax.ShapeDtypeStruct((8, 64), jnp.float32),
        ],
    )(g3, qf, w0, ac0, w1)


def _k5_body(x2_ref, ac_ref, w_ref, xm_ref, st_ref):
    i = pl.program_id(0)
    x2 = x2_ref[...]
    a = ac_ref[0:1, :][None]
    c = ac_ref[1:2, :][None]
    y2 = _gelu(x2 * a + c)
    x3 = _bf16_dot(y2, w_ref[...])
    xm = jnp.max(x3, axis=1)
    xm_ref[...] = xm
    blk = _stats_rows(xm)

    @pl.when(i == 0)
    def _():
        st_ref[...] = blk

    @pl.when(i > 0)
    def _():
        st_ref[...] += blk


def _k5(x2, ac1, w2):
    nq = x2.shape[0]
    return pl.pallas_call(
        _k5_body,
        grid=(nq // _TRK,),
        in_specs=[
            pl.BlockSpec((_TRK, _K, 64), lambda i: (i, 0, 0)),
            pl.BlockSpec((8, 64), lambda i: (0, 0)),
            pl.BlockSpec((128, 64), lambda i: (0, 0)),
        ],
        out_specs=[
            pl.BlockSpec((_TRK, 128), lambda i: (i, 0)),
            pl.BlockSpec((8, 128), lambda i: (0, 0)),
        ],
        out_shape=[
            jax.ShapeDtypeStruct((nq, 128), jnp.float32),
            jax.ShapeDtypeStruct((8, 128), jnp.float32),
        ],
    )(x2, ac1, w2)


def _k6_body(xm_ref, ac_ref, o_ref):
    x = xm_ref[0]
    a = ac_ref[0:1, :]
    c = ac_ref[1:2, :]
    y = _gelu(x * a + c)
    o_ref[0] = jnp.transpose(y)


def _k6(xm3, acn):
    tmo = 512
    return pl.pallas_call(
        _k6_body,
        grid=(_B, _M // tmo),
        in_specs=[
            pl.BlockSpec((1, tmo, 128), lambda b, t: (b, t, 0)),
            pl.BlockSpec((8, 128), lambda b, t: (0, 0)),
        ],
        out_specs=pl.BlockSpec((1, 128, tmo), lambda b, t: (b, 0, t)),
        out_shape=jax.ShapeDtypeStruct((_B, 128, _M), jnp.float32),
    )(xm3, acn)


def _affine(stats, gamma, beta, n):
    s = stats[0]
    s2 = stats[1]
    mean = s / n
    var = s2 / n - mean * mean
    a = gamma * jax.lax.rsqrt(var + _EPS)
    c = beta - mean * a
    pad = jnp.zeros((6, gamma.shape[0]), jnp.float32)
    return jnp.concatenate([a[None, :], c[None, :], pad], axis=0)


def kernel(src_x, src_xyz, xyz, W0, W1, W2, g0, b0, g1, b1, gN, bN):
    tab = _k0(src_x, src_xyz)
    xyzt = jnp.transpose(xyz, (0, 2, 1))
    idx = _k1(xyzt, src_xyz)
    table = tab.reshape(_B * _N, 128)
    flat_idx = idx.reshape(1, _B * _M * _K)
    g = _gather_rows(table, flat_idx)
    g3 = g.reshape(_B * _M, _K, 128)
    qf = xyzt.reshape(_B * _M, 3)

    n0 = float(_B * _M * _K)
    st0 = _k3(g3, qf, W0)
    ac0 = _affine(st0, g0, b0, n0)
    x2, st1 = _k4(g3, qf, W0, ac0, W1)
    ac1 = _affine(st1, g1, b1, n0)
    xm, stn = _k5(x2, ac1, W2)
    acn = _affine(stn, gN, bN, float(_B * _M))
    xm3 = xm.reshape(_B, _M, 128)
    return _k6(xm3, acn)


# speedup vs baseline: 156.1182x; 1.3533x over previous
import math

import jax
import jax.numpy as jnp
from jax.experimental import pallas as pl
from jax.experimental.pallas import tpu as pltpu
from jax.experimental.pallas import tpu_sc as plsc

_B, _N, _M = 2, 8192, 2048
_K = 32
_RADIUS2 = 0.2 * 0.2
_EPS = 1e-5

_TM = 256
_C = 1024
_TRK = 256
_GW = 128


def _gelu(x):
    return 0.5 * x * (1.0 + jax.lax.erf(x * (1.0 / math.sqrt(2.0))))


def _cumsum_lanes(x):
    c = x.shape[-1]
    ii = jax.lax.broadcasted_iota(jnp.int32, x.shape, len(x.shape) - 1)
    s = 1
    while s < c:
        sh = pltpu.roll(x, shift=s, axis=len(x.shape) - 1)
        x = x + jnp.where(ii >= s, sh, 0)
        s *= 2
    return x


def _bf16_dot(lhs, w):
    dn = (((lhs.ndim - 1,), (1,)), ((), ()))
    return jax.lax.dot_general(lhs.astype(jnp.bfloat16),
                               w.astype(jnp.bfloat16), dn,
                               preferred_element_type=jnp.float32)


_NT = 2048


def _k0_body(sx_ref, sxyz_ref, tab_ref):
    xt = jnp.transpose(sx_ref[0])
    zt = jnp.transpose(sxyz_ref[0])
    tab_ref[0] = jnp.concatenate(
        [xt, zt, jnp.zeros((_NT, 61), jnp.float32)], axis=1)


def _k0(src_x, src_xyz):
    return pl.pallas_call(
        _k0_body,
        grid=(_B, _N // _NT),
        in_specs=[
            pl.BlockSpec((1, 64, _NT), lambda b, t: (b, 0, t)),
            pl.BlockSpec((1, 3, _NT), lambda b, t: (b, 0, t)),
        ],
        out_specs=pl.BlockSpec((1, _NT, 128), lambda b, t: (b, t, 0)),
        out_shape=jax.ShapeDtypeStruct((_B, _N, 128), jnp.float32),
    )(src_x, src_xyz)


def _k1_body(q_ref, s_ref, idx_ref):
    b = pl.program_id(0)
    q = q_ref[0]
    qx = q[:, 0:1]
    qy = q[:, 1:2]
    qz = q[:, 2:3]
    q2 = qx * qx + qy * qy + qz * qz
    qxb = qx.astype(jnp.bfloat16).astype(jnp.float32)
    qyb = qy.astype(jnp.bfloat16).astype(jnp.float32)
    qzb = qz.astype(jnp.bfloat16).astype(jnp.float32)

    def chunk(ci, carry):
        cnt, acc = carry

        def live(carry):
            cnt, acc = carry
            s = s_ref[0, :, pl.ds(ci * _C, _C)]
            sx = s[0:1, :]
            sy = s[1:2, :]
            sz = s[2:3, :]
            s2 = sx * sx + sy * sy + sz * sz
            sxb = sx.astype(jnp.bfloat16).astype(jnp.float32)
            syb = sy.astype(jnp.bfloat16).astype(jnp.float32)
            szb = sz.astype(jnp.bfloat16).astype(jnp.float32)
            dot = qxb * sxb + qyb * syb + qzb * szb
            d2 = (q2 + s2) - 2.0 * dot
            mask = d2 <= _RADIUS2
            mi = mask.astype(jnp.int32)
            cum = _cumsum_lanes(mi)
            p = jnp.where(mask, cnt + cum, 0)
            ng = jax.lax.broadcasted_iota(jnp.int32, (_TM, _C), 1) + ci * _C
            cols = []
            for j in range(_K):
                cols.append(jnp.sum(jnp.where(p == (j + 1), ng, 0),
                                    axis=1, keepdims=True))
            acc = acc + jnp.concatenate(cols, axis=1)
            cnt = cnt + cum[:, _C - 1:_C]
            return cnt, acc

        return jax.lax.cond(jnp.min(cnt) >= _K, lambda c: c, live,
                            (cnt, acc))

    cnt0 = jnp.zeros((_TM, 1), jnp.int32)
    acc0 = jnp.zeros((_TM, _K), jnp.int32)
    cnt, acc = jax.lax.fori_loop(0, _N // _C, chunk, (cnt0, acc0))

    jvec = jax.lax.broadcasted_iota(jnp.int32, (_TM, _K), 1)
    filled = jnp.minimum(cnt, _K)
    idx = jnp.where(jvec < filled, acc, acc[:, 0:1])
    idx_ref[0] = idx + b * _N


def _k1(xyzt, src_xyz):
    return pl.pallas_call(
        _k1_body,
        grid=(_B, _M // _TM),
        in_specs=[
            pl.BlockSpec((1, _TM, 3), lambda b, t: (b, t, 0)),
            pl.BlockSpec((1, 3, _N), lambda b, t: (b, 0, 0)),
        ],
        out_specs=pl.BlockSpec((1, _TM, _K), lambda b, t: (b, t, 0)),
        out_shape=jax.ShapeDtypeStruct((_B, _M, _K), jnp.int32),
    )(xyzt, src_xyz)


def _gather_rows(table, indices):
    num = indices.shape[1]
    mesh = plsc.VectorSubcoreMesh(core_axis_name="c", subcore_axis_name="s")

    @pl.kernel(out_type=jax.ShapeDtypeStruct((num, table.shape[1]),
                                             table.dtype),
               mesh=mesh)
    def k(tab_hbm, i_hbm, o_hbm):
        def body(i_vmem, o_vmem):
            pltpu.sync_copy(tab_hbm.at[i_vmem.at[0]], o_vmem)

        pltpu.emit_pipeline(
            body,
            grid=(num // _GW,),
            in_specs=[pl.BlockSpec((1, _GW), index_map=lambda i: (0, i))],
            out_specs=[pl.BlockSpec((_GW, table.shape[1]),
                                    index_map=lambda i: (i, 0))],
            core_axis_name=("c", "s"),
            dimension_semantics=(pltpu.PARALLEL,),
        )(i_hbm, o_hbm)

    return k(table, indices)


def _x1_from_gathered(g_all, q, w0):
    xg = g_all[:, :, :64]
    sg = g_all[:, :, 64:67]
    nx = sg - q[:, None, :]
    feat = jnp.concatenate([xg, nx], axis=2)
    return _bf16_dot(feat, w0)


def _stats_rows(x):
    axes = tuple(range(x.ndim - 1))
    s = jnp.sum(x, axis=axes)
    s2 = jnp.sum(x * x, axis=axes)
    return jnp.concatenate(
        [s[None, :], s2[None, :], jnp.zeros((6, s.shape[0]), jnp.float32)],
        axis=0)


def _k3_body(g_ref, q_ref, w0_ref, st_ref):
    i = pl.program_id(0)
    x1 = _x1_from_gathered(g_ref[...], q_ref[...], w0_ref[...])
    blk = _stats_rows(x1)

    @pl.when(i == 0)
    def _():
        st_ref[...] = blk

    @pl.when(i > 0)
    def _():
        st_ref[...] += blk


def _k3(g3, qf, w0):
    nq = g3.shape[0]
    return pl.pallas_call(
        _k3_body,
        grid=(nq // _TRK,),
        in_specs=[
            pl.BlockSpec((_TRK, _K, 128), lambda i: (i, 0, 0)),
            pl.BlockSpec((_TRK, 3), lambda i: (i, 0)),
            pl.BlockSpec((64, 67), lambda i: (0, 0)),
        ],
        out_specs=pl.BlockSpec((8, 64), lambda i: (0, 0)),
        out_shape=jax.ShapeDtypeStruct((8, 64), jnp.float32),
    )(g3, qf, w0)


def _k4_body(g_ref, q_ref, w0_ref, ac_ref, w1_ref, x2_ref, st_ref):
    i = pl.program_id(0)
    x1 = _x1_from_gathered(g_ref[...], q_ref[...], w0_ref[...])
    a = ac_ref[0:1, :][None]
    c = ac_ref[1:2, :][None]
    y1 = _gelu(x1 * a + c)
    x2 = _bf16_dot(y1, w1_ref[...])
    x2_ref[...] = x2
    blk = _stats_rows(x2)

    @pl.when(i == 0)
    def _():
        st_ref[...] = blk

    @pl.when(i > 0)
    def _():
        st_ref[...] += blk


def _k4(g3, qf, w0, ac0, w1):
    nq = g3.shape[0]
    return pl.pallas_call(
        _k4_body,
        grid=(nq // _TRK,),
        in_specs=[
            pl.BlockSpec((_TRK, _K, 128), lambda i: (i, 0, 0)),
            pl.BlockSpec((_TRK, 3), lambda i: (i, 0)),
            pl.BlockSpec((64, 67), lambda i: (0, 0)),
            pl.BlockSpec((8, 64), lambda i: (0, 0)),
            pl.BlockSpec((64, 64), lambda i: (0, 0)),
        ],
        out_specs=[
            pl.BlockSpec((_TRK, _K, 64), lambda i: (i, 0, 0)),
            pl.BlockSpec((8, 64), lambda i: (0, 0)),
        ],
        out_shape=[
            jax.ShapeDtypeStruct((nq, _K, 64), jnp.float32),
            jax.ShapeDtypeStruct((8, 64), jnp.float32),
        ],
    )(g3, qf, w0, ac0, w1)


def _k5_body(x2_ref, ac_ref, w_ref, xm_ref, st_ref):
    i = pl.program_id(0)
    x2 = x2_ref[...]
    a = ac_ref[0:1, :][None]
    c = ac_ref[1:2, :][None]
    y2 = _gelu(x2 * a + c)
    x3 = _bf16_dot(y2, w_ref[...])
    xm = jnp.max(x3, axis=1)
    xm_ref[...] = xm
    blk = _stats_rows(xm)

    @pl.when(i == 0)
    def _():
        st_ref[...] = blk

    @pl.when(i > 0)
    def _():
        st_ref[...] += blk


def _k5(x2, ac1, w2):
    nq = x2.shape[0]
    return pl.pallas_call(
        _k5_body,
        grid=(nq // _TRK,),
        in_specs=[
            pl.BlockSpec((_TRK, _K, 64), lambda i: (i, 0, 0)),
            pl.BlockSpec((8, 64), lambda i: (0, 0)),
            pl.BlockSpec((128, 64), lambda i: (0, 0)),
        ],
        out_specs=[
            pl.BlockSpec((_TRK, 128), lambda i: (i, 0)),
            pl.BlockSpec((8, 128), lambda i: (0, 0)),
        ],
        out_shape=[
            jax.ShapeDtypeStruct((nq, 128), jnp.float32),
            jax.ShapeDtypeStruct((8, 128), jnp.float32),
        ],
    )(x2, ac1, w2)


def _k6_body(xm_ref, ac_ref, o_ref):
    x = xm_ref[0]
    a = ac_ref[0:1, :]
    c = ac_ref[1:2, :]
    y = _gelu(x * a + c)
    o_ref[0] = jnp.transpose(y)


def _k6(xm3, acn):
    tmo = 512
    return pl.pallas_call(
        _k6_body,
        grid=(_B, _M // tmo),
        in_specs=[
            pl.BlockSpec((1, tmo, 128), lambda b, t: (b, t, 0)),
            pl.BlockSpec((8, 128), lambda b, t: (0, 0)),
        ],
        out_specs=pl.BlockSpec((1, 128, tmo), lambda b, t: (b, 0, t)),
        out_shape=jax.ShapeDtypeStruct((_B, 128, _M), jnp.float32),
    )(xm3, acn)


def _affine(stats, gamma, beta, n):
    s = stats[0]
    s2 = stats[1]
    mean = s / n
    var = s2 / n - mean * mean
    a = gamma * jax.lax.rsqrt(var + _EPS)
    c = beta - mean * a
    pad = jnp.zeros((6, gamma.shape[0]), jnp.float32)
    return jnp.concatenate([a[None, :], c[None, :], pad], axis=0)


def kernel(src_x, src_xyz, xyz, W0, W1, W2, g0, b0, g1, b1, gN, bN):
    tab = _k0(src_x, src_xyz)
    xyzt = jnp.transpose(xyz, (0, 2, 1))
    idx = _k1(xyzt, src_xyz)
    table = tab.reshape(_B * _N, 128)
    flat_idx = idx.reshape(1, _B * _M * _K)
    g = _gather_rows(table, flat_idx)
    g3 = g.reshape(_B * _M, _K, 128)
    qf = xyzt.reshape(_B * _M, 3)

    n0 = float(_B * _M * _K)
    st0 = _k3(g3, qf, W0)
    ac0 = _affine(st0, g0, b0, n0)
    x2, st1 = _k4(g3, qf, W0, ac0, W1)
    ac1 = _affine(st1, g1, b1, n0)
    xm, stn = _k5(x2, ac1, W2)
    acn = _affine(stn, gN, bN, float(_B * _M))
    xm3 = xm.reshape(_B, _M, 128)
    return _k6(xm3, acn)


# speedup vs baseline: 156.7194x; 1.0039x over previous
import math

import jax
import jax.numpy as jnp
from jax.experimental import pallas as pl
from jax.experimental.pallas import tpu as pltpu
from jax.experimental.pallas import tpu_sc as plsc

_B, _N, _M = 2, 8192, 2048
_K = 32
_RADIUS2 = 0.2 * 0.2
_EPS = 1e-5

_TM = 128
_C = 1024
_TRK = 256
_GW = 128


def _gelu(x):
    return 0.5 * x * (1.0 + jax.lax.erf(x * (1.0 / math.sqrt(2.0))))


def _cumsum_lanes(x):
    c = x.shape[-1]
    ii = jax.lax.broadcasted_iota(jnp.int32, x.shape, len(x.shape) - 1)
    s = 1
    while s < c:
        sh = pltpu.roll(x, shift=s, axis=len(x.shape) - 1)
        x = x + jnp.where(ii >= s, sh, 0)
        s *= 2
    return x


def _bf16_dot(lhs, w):
    dn = (((lhs.ndim - 1,), (1,)), ((), ()))
    return jax.lax.dot_general(lhs.astype(jnp.bfloat16),
                               w.astype(jnp.bfloat16), dn,
                               preferred_element_type=jnp.float32)


_NT = 2048


def _k0_body(sx_ref, sxyz_ref, tab_ref):
    xt = jnp.transpose(sx_ref[0])
    zt = jnp.transpose(sxyz_ref[0])
    tab_ref[0] = jnp.concatenate(
        [xt, zt, jnp.zeros((_NT, 61), jnp.float32)], axis=1)


def _k0(src_x, src_xyz):
    return pl.pallas_call(
        _k0_body,
        grid=(_B, _N // _NT),
        in_specs=[
            pl.BlockSpec((1, 64, _NT), lambda b, t: (b, 0, t)),
            pl.BlockSpec((1, 3, _NT), lambda b, t: (b, 0, t)),
        ],
        out_specs=pl.BlockSpec((1, _NT, 128), lambda b, t: (b, t, 0)),
        out_shape=jax.ShapeDtypeStruct((_B, _N, 128), jnp.float32),
    )(src_x, src_xyz)


def _k1_body(q_ref, s_ref, idx_ref):
    b = pl.program_id(0)
    q = q_ref[0]
    qx = q[:, 0:1]
    qy = q[:, 1:2]
    qz = q[:, 2:3]
    q2 = qx * qx + qy * qy + qz * qz
    qxb = qx.astype(jnp.bfloat16).astype(jnp.float32)
    qyb = qy.astype(jnp.bfloat16).astype(jnp.float32)
    qzb = qz.astype(jnp.bfloat16).astype(jnp.float32)

    def chunk(ci, carry):
        cnt, acc = carry

        def live(carry):
            cnt, acc = carry
            s = s_ref[0, :, pl.ds(ci * _C, _C)]
            sx = s[0:1, :]
            sy = s[1:2, :]
            sz = s[2:3, :]
            s2 = sx * sx + sy * sy + sz * sz
            sxb = sx.astype(jnp.bfloat16).astype(jnp.float32)
            syb = sy.astype(jnp.bfloat16).astype(jnp.float32)
            szb = sz.astype(jnp.bfloat16).astype(jnp.float32)
            dot = qxb * sxb + qyb * syb + qzb * szb
            d2 = (q2 + s2) - 2.0 * dot
            mask = d2 <= _RADIUS2
            mi = mask.astype(jnp.int32)
            cum = _cumsum_lanes(mi)
            p = jnp.where(mask, cnt + cum, 0)
            ng = jax.lax.broadcasted_iota(jnp.int32, (_TM, _C), 1) + ci * _C
            cols = []
            for j in range(_K):
                cols.append(jnp.sum(jnp.where(p == (j + 1), ng, 0),
                                    axis=1, keepdims=True))
            acc = acc + jnp.concatenate(cols, axis=1)
            cnt = cnt + cum[:, _C - 1:_C]
            return cnt, acc

        return jax.lax.cond(jnp.min(cnt) >= _K, lambda c: c, live,
                            (cnt, acc))

    cnt0 = jnp.zeros((_TM, 1), jnp.int32)
    acc0 = jnp.zeros((_TM, _K), jnp.int32)
    cnt, acc = jax.lax.fori_loop(0, _N // _C, chunk, (cnt0, acc0))

    jvec = jax.lax.broadcasted_iota(jnp.int32, (_TM, _K), 1)
    filled = jnp.minimum(cnt, _K)
    idx = jnp.where(jvec < filled, acc, acc[:, 0:1])
    idx_ref[0] = idx + b * _N


def _k1(xyzt, src_xyz):
    return pl.pallas_call(
        _k1_body,
        grid=(_B, _M // _TM),
        in_specs=[
            pl.BlockSpec((1, _TM, 3), lambda b, t: (b, t, 0)),
            pl.BlockSpec((1, 3, _N), lambda b, t: (b, 0, 0)),
        ],
        out_specs=pl.BlockSpec((1, _TM, _K), lambda b, t: (b, t, 0)),
        out_shape=jax.ShapeDtypeStruct((_B, _M, _K), jnp.int32),
    )(xyzt, src_xyz)


def _gather_rows(table, indices):
    num = indices.shape[1]
    mesh = plsc.VectorSubcoreMesh(core_axis_name="c", subcore_axis_name="s")

    @pl.kernel(out_type=jax.ShapeDtypeStruct((num, table.shape[1]),
                                             table.dtype),
               mesh=mesh)
    def k(tab_hbm, i_hbm, o_hbm):
        def body(i_vmem, o_vmem):
            pltpu.sync_copy(tab_hbm.at[i_vmem.at[0]], o_vmem)

        pltpu.emit_pipeline(
            body,
            grid=(num // _GW,),
            in_specs=[pl.BlockSpec((1, _GW), index_map=lambda i: (0, i))],
            out_specs=[pl.BlockSpec((_GW, table.shape[1]),
                                    index_map=lambda i: (i, 0))],
            core_axis_name=("c", "s"),
            dimension_semantics=(pltpu.PARALLEL,),
        )(i_hbm, o_hbm)

    return k(table, indices)


def _x1_from_gathered(g_all, q, w0):
    xg = g_all[:, :, :64]
    sg = g_all[:, :, 64:67]
    nx = sg - q[:, None, :]
    feat = jnp.concatenate([xg, nx], axis=2)
    return _bf16_dot(feat, w0)


def _stats_rows(x):
    axes = tuple(range(x.ndim - 1))
    s = jnp.sum(x, axis=axes)
    s2 = jnp.sum(x * x, axis=axes)
    return jnp.concatenate(
        [s[None, :], s2[None, :], jnp.zeros((6, s.shape[0]), jnp.float32)],
        axis=0)


def _k3_body(g_ref, q_ref, w0_ref, st_ref):
    i = pl.program_id(0)
    x1 = _x1_from_gathered(g_ref[...], q_ref[...], w0_ref[...])
    blk = _stats_rows(x1)

    @pl.when(i == 0)
    def _():
        st_ref[...] = blk

    @pl.when(i > 0)
    def _():
        st_ref[...] += blk


def _k3(g3, qf, w0):
    nq = g3.shape[0]
    return pl.pallas_call(
        _k3_body,
        grid=(nq // _TRK,),
        in_specs=[
            pl.BlockSpec((_TRK, _K, 128), lambda i: (i, 0, 0)),
            pl.BlockSpec((_TRK, 3), lambda i: (i, 0)),
            pl.BlockSpec((64, 67), lambda i: (0, 0)),
        ],
        out_specs=pl.BlockSpec((8, 64), lambda i: (0, 0)),
        out_shape=jax.ShapeDtypeStruct((8, 64), jnp.float32),
    )(g3, qf, w0)


def _k4_body(g_ref, q_ref, w0_ref, ac_ref, w1_ref, x2_ref, st_ref):
    i = pl.program_id(0)
    x1 = _x1_from_gathered(g_ref[...], q_ref[...], w0_ref[...])
    a = ac_ref[0:1, :][None]
    c = ac_ref[1:2, :][None]
    y1 = _gelu(x1 * a + c)
    x2 = _bf16_dot(y1, w1_ref[...])
    x2_ref[...] = x2
    blk = _stats_rows(x2)

    @pl.when(i == 0)
    def _():
        st_ref[...] = blk

    @pl.when(i > 0)
    def _():
        st_ref[...] += blk


def _k4(g3, qf, w0, ac0, w1):
    nq = g3.shape[0]
    return pl.pallas_call(
        _k4_body,
        grid=(nq // _TRK,),
        in_specs=[
            pl.BlockSpec((_TRK, _K, 128), lambda i: (i, 0, 0)),
            pl.BlockSpec((_TRK, 3), lambda i: (i, 0)),
            pl.BlockSpec((64, 67), lambda i: (0, 0)),
            pl.BlockSpec((8, 64), lambda i: (0, 0)),
            pl.BlockSpec((64, 64), lambda i: (0, 0)),
        ],
        out_specs=[
            pl.BlockSpec((_TRK, _K, 64), lambda i: (i, 0, 0)),
            pl.BlockSpec((8, 64), lambda i: (0, 0)),
        ],
        out_shape=[
            jax.ShapeDtypeStruct((nq, _K, 64), jnp.float32),
            jax.ShapeDtypeStruct((8, 64), jnp.float32),
        ],
    )(g3, qf, w0, ac0, w1)


def _k5_body(x2_ref, ac_ref, w_ref, xm_ref, st_ref):
    i = pl.program_id(0)
    x2 = x2_ref[...]
    a = ac_ref[0:1, :][None]
    c = ac_ref[1:2, :][None]
    y2 = _gelu(x2 * a + c)
    x3 = _bf16_dot(y2, w_ref[...])
    xm = jnp.max(x3, axis=1)
    xm_ref[...] = xm
    blk = _stats_rows(xm)

    @pl.when(i == 0)
    def _():
        st_ref[...] = blk

    @pl.when(i > 0)
    def _():
        st_ref[...] += blk


def _k5(x2, ac1, w2):
    nq = x2.shape[0]
    return pl.pallas_call(
        _k5_body,
        grid=(nq // _TRK,),
        in_specs=[
            pl.BlockSpec((_TRK, _K, 64), lambda i: (i, 0, 0)),
            pl.BlockSpec((8, 64), lambda i: (0, 0)),
            pl.BlockSpec((128, 64), lambda i: (0, 0)),
        ],
        out_specs=[
            pl.BlockSpec((_TRK, 128), lambda i: (i, 0)),
            pl.BlockSpec((8, 128), lambda i: (0, 0)),
        ],
        out_shape=[
            jax.ShapeDtypeStruct((nq, 128), jnp.float32),
            jax.ShapeDtypeStruct((8, 128), jnp.float32),
        ],
    )(x2, ac1, w2)


def _k6_body(xm_ref, ac_ref, o_ref):
    x = xm_ref[0]
    a = ac_ref[0:1, :]
    c = ac_ref[1:2, :]
    y = _gelu(x * a + c)
    o_ref[0] = jnp.transpose(y)


def _k6(xm3, acn):
    tmo = 512
    return pl.pallas_call(
        _k6_body,
        grid=(_B, _M // tmo),
        in_specs=[
            pl.BlockSpec((1, tmo, 128), lambda b, t: (b, t, 0)),
            pl.BlockSpec((8, 128), lambda b, t: (0, 0)),
        ],
        out_specs=pl.BlockSpec((1, 128, tmo), lambda b, t: (b, 0, t)),
        out_shape=jax.ShapeDtypeStruct((_B, 128, _M), jnp.float32),
    )(xm3, acn)


def _affine(stats, gamma, beta, n):
    s = stats[0]
    s2 = stats[1]
    mean = s / n
    var = s2 / n - mean * mean
    a = gamma * jax.lax.rsqrt(var + _EPS)
    c = beta - mean * a
    pad = jnp.zeros((6, gamma.shape[0]), jnp.float32)
    return jnp.concatenate([a[None, :], c[None, :], pad], axis=0)


def kernel(src_x, src_xyz, xyz, W0, W1, W2, g0, b0, g1, b1, gN, bN):
    tab = _k0(src_x, src_xyz)
    xyzt = jnp.transpose(xyz, (0, 2, 1))
    idx = _k1(xyzt, src_xyz)
    table = tab.reshape(_B * _N, 128)
    flat_idx = idx.reshape(1, _B * _M * _K)
    g = _gather_rows(table, flat_idx)
    g3 = g.reshape(_B * _M, _K, 128)
    qf = xyzt.reshape(_B * _M, 3)

    n0 = float(_B * _M * _K)
    st0 = _k3(g3, qf, W0)
    ac0 = _affine(st0, g0, b0, n0)
    x2, st1 = _k4(g3, qf, W0, ac0, W1)
    ac1 = _affine(st1, g1, b1, n0)
    xm, stn = _k5(x2, ac1, W2)
    acn = _affine(stn, gN, bN, float(_B * _M))
    xm3 = xm.reshape(_B, _M, 128)
    return _k6(xm3, acn)


# speedup vs baseline: 162.0052x; 1.0337x over previous
import functools
import math

import jax
import jax.numpy as jnp
from jax.experimental import pallas as pl
from jax.experimental.pallas import tpu as pltpu
from jax.experimental.pallas import tpu_sc as plsc

_B, _N, _M = 2, 8192, 2048
_K = 32
_RADIUS2 = 0.2 * 0.2
_EPS = 1e-5

_TM = 128
_C = 1024
_TRK = 256
_GW = 128


def _gelu(x):
    return 0.5 * x * (1.0 + jax.lax.erf(x * (1.0 / math.sqrt(2.0))))


def _cumsum_lanes(x):
    c = x.shape[-1]
    ii = jax.lax.broadcasted_iota(jnp.int32, x.shape, len(x.shape) - 1)
    s = 1
    while s < c:
        sh = pltpu.roll(x, shift=s, axis=len(x.shape) - 1)
        x = x + jnp.where(ii >= s, sh, 0)
        s *= 2
    return x


def _bf16_dot(lhs, w):
    dn = (((lhs.ndim - 1,), (1,)), ((), ()))
    return jax.lax.dot_general(lhs.astype(jnp.bfloat16),
                               w.astype(jnp.bfloat16), dn,
                               preferred_element_type=jnp.float32)


_NT = 2048


def _k0_body(sx_ref, sxyz_ref, tab_ref):
    xt = jnp.transpose(sx_ref[0])
    zt = jnp.transpose(sxyz_ref[0])
    tab_ref[0] = jnp.concatenate(
        [xt, zt, jnp.zeros((_NT, 61), jnp.float32)], axis=1)


def _k0(src_x, src_xyz):
    return pl.pallas_call(
        _k0_body,
        grid=(_B, _N // _NT),
        in_specs=[
            pl.BlockSpec((1, 64, _NT), lambda b, t: (b, 0, t)),
            pl.BlockSpec((1, 3, _NT), lambda b, t: (b, 0, t)),
        ],
        out_specs=pl.BlockSpec((1, _NT, 128), lambda b, t: (b, t, 0)),
        out_shape=jax.ShapeDtypeStruct((_B, _N, 128), jnp.float32),
    )(src_x, src_xyz)


def _k1_body(bo, q_ref, s_ref, idx_ref):
    b = pl.program_id(0) + bo
    q = q_ref[0]
    qx = q[:, 0:1]
    qy = q[:, 1:2]
    qz = q[:, 2:3]
    q2 = qx * qx + qy * qy + qz * qz
    qxb = qx.astype(jnp.bfloat16).astype(jnp.float32)
    qyb = qy.astype(jnp.bfloat16).astype(jnp.float32)
    qzb = qz.astype(jnp.bfloat16).astype(jnp.float32)

    def chunk(ci, carry):
        cnt, acc = carry

        def live(carry):
            cnt, acc = carry
            s = s_ref[0, :, pl.ds(ci * _C, _C)]
            sx = s[0:1, :]
            sy = s[1:2, :]
            sz = s[2:3, :]
            s2 = sx * sx + sy * sy + sz * sz
            sxb = sx.astype(jnp.bfloat16).astype(jnp.float32)
            syb = sy.astype(jnp.bfloat16).astype(jnp.float32)
            szb = sz.astype(jnp.bfloat16).astype(jnp.float32)
            dot = qxb * sxb + qyb * syb + qzb * szb
            d2 = (q2 + s2) - 2.0 * dot
            mask = d2 <= _RADIUS2
            mi = mask.astype(jnp.int32)
            cum = _cumsum_lanes(mi)
            p = jnp.where(mask, cnt + cum, 0)
            ng = jax.lax.broadcasted_iota(jnp.int32, (_TM, _C), 1) + ci * _C
            cols = []
            for j in range(_K):
                cols.append(jnp.sum(jnp.where(p == (j + 1), ng, 0),
                                    axis=1, keepdims=True))
            acc = acc + jnp.concatenate(cols, axis=1)
            cnt = cnt + cum[:, _C - 1:_C]
            return cnt, acc

        return jax.lax.cond(jnp.min(cnt) >= _K, lambda c: c, live,
                            (cnt, acc))

    cnt0 = jnp.zeros((_TM, 1), jnp.int32)
    acc0 = jnp.zeros((_TM, _K), jnp.int32)
    cnt, acc = jax.lax.fori_loop(0, _N // _C, chunk, (cnt0, acc0))

    jvec = jax.lax.broadcasted_iota(jnp.int32, (_TM, _K), 1)
    filled = jnp.minimum(cnt, _K)
    idx = jnp.where(jvec < filled, acc, acc[:, 0:1])
    idx_ref[0] = idx + b * _N


def _k1(xyzt, src_xyz, bo):
    nb = xyzt.shape[0]
    return pl.pallas_call(
        functools.partial(_k1_body, bo),
        grid=(nb, _M // _TM),
        in_specs=[
            pl.BlockSpec((1, _TM, 3), lambda b, t: (b, t, 0)),
            pl.BlockSpec((1, 3, _N), lambda b, t: (b, 0, 0)),
        ],
        out_specs=pl.BlockSpec((1, _TM, _K), lambda b, t: (b, t, 0)),
        out_shape=jax.ShapeDtypeStruct((nb, _M, _K), jnp.int32),
    )(xyzt, src_xyz)


def _gather_rows(table, indices):
    num = indices.shape[1]
    mesh = plsc.VectorSubcoreMesh(core_axis_name="c", subcore_axis_name="s")

    @pl.kernel(out_type=jax.ShapeDtypeStruct((num, table.shape[1]),
                                             table.dtype),
               mesh=mesh)
    def k(tab_hbm, i_hbm, o_hbm):
        def body(i_vmem, o_vmem):
            pltpu.sync_copy(tab_hbm.at[i_vmem.at[0]], o_vmem)

        pltpu.emit_pipeline(
            body,
            grid=(num // _GW,),
            in_specs=[pl.BlockSpec((1, _GW), index_map=lambda i: (0, i))],
            out_specs=[pl.BlockSpec((_GW, table.shape[1]),
                                    index_map=lambda i: (i, 0))],
            core_axis_name=("c", "s"),
            dimension_semantics=(pltpu.PARALLEL,),
        )(i_hbm, o_hbm)

    return k(table, indices)


def _x1_from_gathered(g_all, q, w0):
    xg = g_all[:, :, :64]
    sg = g_all[:, :, 64:67]
    nx = sg - q[:, None, :]
    feat = jnp.concatenate([xg, nx], axis=2)
    return _bf16_dot(feat, w0)


def _stats_rows(x):
    axes = tuple(range(x.ndim - 1))
    s = jnp.sum(x, axis=axes)
    s2 = jnp.sum(x * x, axis=axes)
    return jnp.concatenate(
        [s[None, :], s2[None, :], jnp.zeros((6, s.shape[0]), jnp.float32)],
        axis=0)


def _k3_body(g_ref, q_ref, w0_ref, st_ref):
    i = pl.program_id(0)
    x1 = _x1_from_gathered(g_ref[...], q_ref[...], w0_ref[...])
    blk = _stats_rows(x1)

    @pl.when(i == 0)
    def _():
        st_ref[...] = blk

    @pl.when(i > 0)
    def _():
        st_ref[...] += blk


def _k3(g3, qf, w0):
    nq = g3.shape[0]
    return pl.pallas_call(
        _k3_body,
        grid=(nq // _TRK,),
        in_specs=[
            pl.BlockSpec((_TRK, _K, 128), lambda i: (i, 0, 0)),
            pl.BlockSpec((_TRK, 3), lambda i: (i, 0)),
            pl.BlockSpec((64, 67), lambda i: (0, 0)),
        ],
        out_specs=pl.BlockSpec((8, 64), lambda i: (0, 0)),
        out_shape=jax.ShapeDtypeStruct((8, 64), jnp.float32),
    )(g3, qf, w0)


def _k4_body(g_ref, q_ref, w0_ref, ac_ref, w1_ref, x2_ref, st_ref):
    i = pl.program_id(0)
    x1 = _x1_from_gathered(g_ref[...], q_ref[...], w0_ref[...])
    a = ac_ref[0:1, :][None]
    c = ac_ref[1:2, :][None]
    y1 = _gelu(x1 * a + c)
    x2 = _bf16_dot(y1, w1_ref[...])
    x2_ref[...] = x2
    blk = _stats_rows(x2)

    @pl.when(i == 0)
    def _():
        st_ref[...] = blk

    @pl.when(i > 0)
    def _():
        st_ref[...] += blk


def _k4(g3, qf, w0, ac0, w1):
    nq = g3.shape[0]
    return pl.pallas_call(
        _k4_body,
        grid=(nq // _TRK,),
        in_specs=[
            pl.BlockSpec((_TRK, _K, 128), lambda i: (i, 0, 0)),
            pl.BlockSpec((_TRK, 3), lambda i: (i, 0)),
            pl.BlockSpec((64, 67), lambda i: (0, 0)),
            pl.BlockSpec((8, 64), lambda i: (0, 0)),
            pl.BlockSpec((64, 64), lambda i: (0, 0)),
        ],
        out_specs=[
            pl.BlockSpec((_TRK, _K, 64), lambda i: (i, 0, 0)),
            pl.BlockSpec((8, 64), lambda i: (0, 0)),
        ],
        out_shape=[
            jax.ShapeDtypeStruct((nq, _K, 64), jnp.float32),
            jax.ShapeDtypeStruct((8, 64), jnp.float32),
        ],
    )(g3, qf, w0, ac0, w1)


def _k5_body(x2_ref, ac_ref, w_ref, xm_ref, st_ref):
    i = pl.program_id(0)
    x2 = x2_ref[...]
    a = ac_ref[0:1, :][None]
    c = ac_ref[1:2, :][None]
    y2 = _gelu(x2 * a + c)
    x3 = _bf16_dot(y2, w_ref[...])
    xm = jnp.max(x3, axis=1)
    xm_ref[...] = xm
    blk = _stats_rows(xm)

    @pl.when(i == 0)
    def _():
        st_ref[...] = blk

    @pl.when(i > 0)
    def _():
        st_ref[...] += blk


def _k5(x2, ac1, w2):
    nq = x2.shape[0]
    return pl.pallas_call(
        _k5_body,
        grid=(nq // _TRK,),
        in_specs=[
            pl.BlockSpec((_TRK, _K, 64), lambda i: (i, 0, 0)),
            pl.BlockSpec((8, 64), lambda i: (0, 0)),
            pl.BlockSpec((128, 64), lambda i: (0, 0)),
        ],
        out_specs=[
            pl.BlockSpec((_TRK, 128), lambda i: (i, 0)),
            pl.BlockSpec((8, 128), lambda i: (0, 0)),
        ],
        out_shape=[
            jax.ShapeDtypeStruct((nq, 128), jnp.float32),
            jax.ShapeDtypeStruct((8, 128), jnp.float32),
        ],
    )(x2, ac1, w2)


def _k6_body(xm_ref, ac_ref, o_ref):
    x = xm_ref[0]
    a = ac_ref[0:1, :]
    c = ac_ref[1:2, :]
    y = _gelu(x * a + c)
    o_ref[0] = jnp.transpose(y)


def _k6(xm3, acn):
    tmo = 512
    return pl.pallas_call(
        _k6_body,
        grid=(_B, _M // tmo),
        in_specs=[
            pl.BlockSpec((1, tmo, 128), lambda b, t: (b, t, 0)),
            pl.BlockSpec((8, 128), lambda b, t: (0, 0)),
        ],
        out_specs=pl.BlockSpec((1, 128, tmo), lambda b, t: (b, 0, t)),
        out_shape=jax.ShapeDtypeStruct((_B, 128, _M), jnp.float32),
    )(xm3, acn)


def _affine(stats, gamma, beta, n):
    s = stats[0]
    s2 = stats[1]
    mean = s / n
    var = s2 / n - mean * mean
    a = gamma * jax.lax.rsqrt(var + _EPS)
    c = beta - mean * a
    pad = jnp.zeros((6, gamma.shape[0]), jnp.float32)
    return jnp.concatenate([a[None, :], c[None, :], pad], axis=0)


def kernel(src_x, src_xyz, xyz, W0, W1, W2, g0, b0, g1, b1, gN, bN):
    tab = _k0(src_x, src_xyz)
    xyzt = jnp.transpose(xyz, (0, 2, 1))
    table = tab.reshape(_B * _N, 128)

    g3 = []
    qh = []
    for b in range(_B):
        idx_b = _k1(xyzt[b:b + 1], src_xyz[b:b + 1], b)
        fi_b = idx_b.reshape(1, _M * _K)
        g_b = _gather_rows(table, fi_b)
        g3.append(g_b.reshape(_M, _K, 128))
        qh.append(xyzt[b].reshape(_M, 3))

    n0 = float(_B * _M * _K)
    st0 = _k3(g3[0], qh[0], W0) + _k3(g3[1], qh[1], W0)
    ac0 = _affine(st0, g0, b0, n0)
    x2a, st1a = _k4(g3[0], qh[0], W0, ac0, W1)
    x2b, st1b = _k4(g3[1], qh[1], W0, ac0, W1)
    ac1 = _affine(st1a + st1b, g1, b1, n0)
    xma, stna = _k5(x2a, ac1, W2)
    xmb, stnb = _k5(x2b, ac1, W2)
    acn = _affine(stna + stnb, gN, bN, float(_B * _M))
    xm3 = jnp.concatenate(
        [xma.reshape(1, _M, 128), xmb.reshape(1, _M, 128)], axis=0)
    return _k6(xm3, acn)
